# baseline (speedup 1.0000x reference)
import functools
import math

import jax
import jax.numpy as jnp
from jax import lax
from jax.experimental import pallas as pl
from jax.experimental.pallas import tpu as pltpu
from jax.experimental.pallas import tpu_sc as plsc

D_KK = 64
N_ROUNDS = 4
BKT = 64
SEQ = 4096
NBKT = SEQ // BKT
NBB = 8
NBLK = NBKT // NBB
CHUNK = 512


def _hash_rank_body(qref, rmref, oiref, oigref, sqhref, sbpref):
    q = qref[0]
    rm = rmref[0, 0]
    b = pl.program_id(0)
    r = pl.program_id(1)
    nrm = jnp.sqrt(jnp.sum(q * q, axis=1, keepdims=True))
    qn = q / jnp.maximum(nrm, 1e-12)
    m = jax.lax.dot_general(
        qn, rm, (((1,), (0,)), ((), ())), preferred_element_type=jnp.float32)

    L = SEQ
    lane64 = jax.lax.broadcasted_iota(jnp.int32, (L, NBKT), 1)
    row_iota = jax.lax.broadcasted_iota(jnp.int32, (L, 1), 0)
    lane64h = jax.lax.broadcasted_iota(jnp.int32, (1, NBKT), 1)
    lt = (jax.lax.broadcasted_iota(jnp.int32, (128, 128), 0) >=
          jax.lax.broadcasted_iota(jnp.int32, (128, 128), 1)
          ).astype(jnp.float32)
    c = jnp.concatenate([m, -m], axis=1)
    mx = jnp.max(c, axis=1, keepdims=True)
    hv = jnp.min(jnp.where(c == mx, lane64, NBKT), axis=1,
                 keepdims=True)

    onehot = (hv == lane64).astype(jnp.float32)
    pieces = []
    carry = jnp.zeros((1, NBKT), jnp.float32)
    for ch in range(L // 128):
        oh_c = onehot[ch * 128:(ch + 1) * 128, :]
        within = jax.lax.dot_general(
            lt, oh_c, (((1,), (0,)), ((), ())),
            preferred_element_type=jnp.float32)
        pieces.append(within + carry)
        carry = carry + within[127:128, :]
    ccum = jnp.concatenate(pieces, axis=0)
    c_incl = jnp.sum(onehot * ccum, axis=1, keepdims=True)
    tot = carry
    cumt = tot
    k = 1
    while k < NBKT:
        sh2 = jnp.where(lane64h < k, 0.0, jnp.roll(cumt, k, axis=1))
        cumt = cumt + sh2
        k *= 2
    offs = cumt - tot
    oi_f = jnp.sum(onehot * offs, axis=1, keepdims=True) + c_incl - 1.0
    oi = oi_f.astype(jnp.int32)

    sqh = jnp.sum(
        (cumt <= row_iota.astype(jnp.float32)).astype(jnp.int32),
        axis=1, keepdims=True)

    oiref[0, 0] = oi
    oigref[0, 0] = oi + (b * N_ROUNDS + r) * L
    sqhref[0, 0] = sqh
    part = (oi >> 6) << (8 * r)
    sbpref[0] = jnp.where(r == 0, part, sbpref[0] | part)


def _sc_reorder_body(qv_hbm, oi_hbm, oig_hbm, sbp_hbm,
                     rqv_hbm, qi_hbm, sbq_hbm,
                     oi_v, idx_v, qvbuf, qi_b, sbq_b, sbp_v, sem):
    info = plsc.get_sparse_core_info()
    b = lax.axis_index("s") * info.num_cores + lax.axis_index("c")
    L = SEQ
    pltpu.sync_copy(sbp_hbm.at[b], sbp_v)
    for r in range(N_ROUNDS):
        pltpu.sync_copy(oi_hbm.at[b, r], oi_v)

        def body(i, carry):
            sl = pl.ds(i * 16, 16)
            idx16 = oi_v[sl]
            lvals = i * 16 + lax.iota(jnp.int32, 16)
            plsc.store_scatter(qi_b, [idx16], lvals)
            plsc.store_scatter(sbq_b, [idx16], sbp_v[sl])
            return carry

        lax.fori_loop(0, L // 16, body, 0)
        pltpu.sync_copy(qi_b, qi_hbm.at[b, r])
        pltpu.sync_copy(sbq_b, sbq_hbm.at[b, r])
    nsub = CHUNK // 128
    for c in range(L // CHUNK):
        pltpu.sync_copy(qv_hbm.at[b, pl.ds(c * CHUNK, CHUNK)], qvbuf)
        for r in range(N_ROUNDS):
            pltpu.sync_copy(oig_hbm.at[b, r, pl.ds(c * nsub, nsub)], idx_v)
            for j in range(nsub):
                pltpu.async_copy(qvbuf.at[pl.ds(j * 128, 128)],
                                 rqv_hbm.at[idx_v.at[j]], sem).wait()


def _sc_restore_body(att_hbm, oig_hbm, atto_hbm, idx_v, rows_v, sem):
    info = plsc.get_sparse_core_info()
    b = lax.axis_index("s") * info.num_cores + lax.axis_index("c")
    L = SEQ
    nsub = CHUNK // 128
    for r in range(N_ROUNDS):
        for c in range(L // CHUNK):
            pltpu.sync_copy(oig_hbm.at[b, r, pl.ds(c * nsub, nsub)], idx_v)
            for j in range(nsub):
                pltpu.async_copy(att_hbm.at[idx_v.at[j]],
                                 rows_v.at[pl.ds(j * 128, 128)], sem).wait()
            pltpu.sync_copy(rows_v, atto_hbm.at[b, r, pl.ds(c * CHUNK, CHUNK)])


def _attn_body(rqv, rqvh, qik, qikh, qic, sqhk, sqhkh, sqhc,
               sbqk, sbqkh, sbqc, att):
    qv = rqv[0, 0]
    qvh = rqvh[0, 0]
    q = qv[:, :D_KK]
    qh = qvh[:, :D_KK]
    v = qv[:, D_KK:]
    vh = qvh[:, D_KK:]
    qi_rows = qik[0, 0, :, 0]
    qi_prev0 = qikh[0, 0, 0]
    qi_col = qic[0, 0]
    sq_rows = sqhk[0, 0, :, 0]
    sq_prev0 = sqhkh[0, 0, 0]
    sq_col = sqhc[0, 0]
    sb_rows = sbqk[0, 0, :, 0]
    sb_prev0 = sbqkh[0, 0, 0]
    sb_col = sbqc[0, 0]

    inv_sqrt_d = 1.0 / math.sqrt(D_KK)
    qs = q * inv_sqrt_d
    qhs = qh * inv_sqrt_d
    nrm = jnp.sqrt(jnp.sum(qs * qs, axis=1, keepdims=True))
    kna = qs / jnp.maximum(nrm, 1e-12 * inv_sqrt_d)
    nrmh = jnp.sqrt(jnp.sum(qhs * qhs, axis=1, keepdims=True))
    knh = qhs / jnp.maximum(nrmh, 1e-12 * inv_sqrt_d)
    sbu_col = [(sb_col >> (8 * rp)) & 255 for rp in range(N_ROUNDS)]
    sbu_rows = [(sb_rows >> (8 * rp)) & 255 for rp in range(N_ROUNDS)]
    sbu_prev0 = [(sb_prev0 >> (8 * rp)) & 255 for rp in range(N_ROUNDS)]
    for t in range(NBB):
        qt = qs[t * BKT:(t + 1) * BKT, :]
        knp = knh if t == 0 else kna[(t - 1) * BKT:t * BKT, :]
        kn = jnp.concatenate([knp, kna[t * BKT:(t + 1) * BKT, :]], axis=0)
        s = jax.lax.dot_general(
            qt, kn, (((1,), (1,)), ((), ())),
            preferred_element_type=jnp.float32)

        qi_q = qi_col[t * BKT:(t + 1) * BKT, :]
        qi_p = qi_prev0 if t == 0 else qi_rows[t - 1:t, :]
        qi_k = jnp.concatenate([qi_p, qi_rows[t:t + 1, :]], axis=1)
        sq_q = sq_col[t * BKT:(t + 1) * BKT, :]
        sq_p = sq_prev0 if t == 0 else sq_rows[t - 1:t, :]
        sq_k = jnp.concatenate([sq_p, sq_rows[t:t + 1, :]], axis=1)

        s = jnp.where((sq_q != sq_k) | (qi_q < qi_k), -1e9, s)
        s = jnp.where(qi_q == qi_k, -1e5, s)

        cnt = jnp.zeros(s.shape, jnp.int32)
        for rp in range(N_ROUNDS):
            aa = sbu_col[rp][t * BKT:(t + 1) * BKT, :]
            bp = sbu_prev0[rp] if t == 0 else sbu_rows[rp][t - 1:t, :]
            bb = jnp.concatenate([bp, sbu_rows[rp][t:t + 1, :]], axis=1)
            cnt = cnt + jnp.where(((aa - bb) & (NBKT - 1)) <= 1, 1, 0)

        m = jnp.max(s, axis=1, keepdims=True)
        e = jnp.exp(s - m)
        ssum = jnp.sum(e, axis=1, keepdims=True)
        lse_t = m + jnp.log(ssum)
        rcp = jnp.where(cnt == 1, 1.0,
                        jnp.where(cnt == 2, 0.5,
                                  jnp.where(cnt == 3, 1.0 / 3.0, 0.25)))
        p = e * rcp * (1.0 / ssum)

        vw = jnp.concatenate([vh if t == 0 else v[(t - 1) * BKT:t * BKT, :],
                              v[t * BKT:(t + 1) * BKT, :]], axis=0)
        ot = jax.lax.dot_general(
            p, vw, (((1,), (0,)), ((), ())),
            preferred_element_type=jnp.float32)
        att[0, 0, t * BKT:(t + 1) * BKT, 0:D_KK] = ot
        att[0, 0, t * BKT:(t + 1) * BKT, D_KK:D_KK + 1] = lse_t


def _combine_body(att_ref, out_ref):
    a = att_ref[0]
    acc = None
    for r in range(N_ROUNDS):
        x = a[r, :, D_KK:D_KK + 1]
        m = jnp.max(x)
        e = jnp.exp(x - m)
        w = e / jnp.sum(e)
        term = a[r, :, :D_KK] * w
        acc = term if acc is None else acc + term
    out_ref[0] = acc


@jax.jit
def kernel(query, value, rand_matrix, seed):
    B, L, D = query.shape
    R = N_ROUNDS

    rm2 = rand_matrix.reshape(B, D, R, NBKT // 2).transpose(0, 2, 1, 3)
    oi4, oig4, sqh4, sbp3 = pl.pallas_call(
        _hash_rank_body,
        grid=(B, R),
        in_specs=[
            pl.BlockSpec((1, L, D), lambda b, r: (b, 0, 0)),
            pl.BlockSpec((1, 1, D, NBKT // 2), lambda b, r: (b, r, 0, 0)),
        ],
        out_specs=[
            pl.BlockSpec((1, 1, L, 1), lambda b, r: (b, r, 0, 0)),
            pl.BlockSpec((1, 1, L, 1), lambda b, r: (b, r, 0, 0)),
            pl.BlockSpec((1, 1, L, 1), lambda b, r: (b, r, 0, 0)),
            pl.BlockSpec((1, L, 1), lambda b, r: (b, 0, 0)),
        ],
        out_shape=[
            jax.ShapeDtypeStruct((B, R, L, 1), jnp.int32),
            jax.ShapeDtypeStruct((B, R, L, 1), jnp.int32),
            jax.ShapeDtypeStruct((B, R, L, 1), jnp.int32),
            jax.ShapeDtypeStruct((B, L, 1), jnp.int32),
        ],
    )(query, rm2)

    oi3 = oi4.reshape(B, R, L)
    oig3 = oig4.reshape(B, R, L)
    sbp2 = sbp3.reshape(B, L)

    qv = jnp.concatenate([query, value], axis=-1)
    mesh = plsc.VectorSubcoreMesh(core_axis_name="c", subcore_axis_name="s")
    rqvf, qi3, sbq3 = pl.kernel(
        _sc_reorder_body,
        mesh=mesh,
        compiler_params=pltpu.CompilerParams(needs_layout_passes=False),
        out_type=[
            jax.ShapeDtypeStruct((B * R * L, 2 * D), jnp.float32),
            jax.ShapeDtypeStruct((B, R, L), jnp.int32),
            jax.ShapeDtypeStruct((B, R, L), jnp.int32),
        ],
        scratch_types=[
            pltpu.VMEM((L,), jnp.int32),
            pltpu.VMEM((CHUNK // 128, 128), jnp.int32),
            pltpu.VMEM((CHUNK, 2 * D), jnp.float32),
            pltpu.VMEM((L,), jnp.int32),
            pltpu.VMEM((L,), jnp.int32),
            pltpu.VMEM((L,), jnp.int32),
            pltpu.SemaphoreType.DMA,
        ],
    )(qv, oi3, oig4.reshape(B, R, L // 128, 128), sbp2)

    rqvg = rqvf.reshape(B, R, L, 2 * D)

    qik = qi3.reshape(B, R, NBKT, 1, BKT)
    sqhk = sqh4.reshape(B, R, NBKT, 1, BKT)
    sbqk = sbq3.reshape(B, R, NBKT, 1, BKT)
    qic = qi3.reshape(B, R, L, 1)
    sqhc = sqh4.reshape(B, R, L, 1)
    sbqc = sbq3.reshape(B, R, L, 1)

    grid = (B, R, NBLK)

    def blk(b, r, n):
        return (b, r, n, 0)

    def blk_halo(b, r, n):
        return (b, r, (n * NBB - 1) % NBKT, 0)

    def blk5(b, r, n):
        return (b, r, n, 0, 0)

    def blk5_halo(b, r, n):
        return (b, r, (n * NBB - 1) % NBKT, 0, 0)

    att = pl.pallas_call(
        _attn_body,
        grid=grid,
        in_specs=[
            pl.BlockSpec((1, 1, NBB * BKT, 2 * D), blk),
            pl.BlockSpec((1, 1, BKT, 2 * D), blk_halo),
            pl.BlockSpec((1, 1, NBB, 1, BKT), blk5),
            pl.BlockSpec((1, 1, 1, 1, BKT), blk5_halo),
            pl.BlockSpec((1, 1, NBB * BKT, 1), blk),
            pl.BlockSpec((1, 1, NBB, 1, BKT), blk5),
            pl.BlockSpec((1, 1, 1, 1, BKT), blk5_halo),
            pl.BlockSpec((1, 1, NBB * BKT, 1), blk),
            pl.BlockSpec((1, 1, NBB, 1, BKT), blk5),
            pl.BlockSpec((1, 1, 1, 1, BKT), blk5_halo),
            pl.BlockSpec((1, 1, NBB * BKT, 1), blk),
        ],
        out_specs=pl.BlockSpec((1, 1, NBB * BKT, 2 * D), blk),
        out_shape=jax.ShapeDtypeStruct((B, R, L, 2 * D), jnp.float32),
    )(rqvg, rqvg, qik, qik, qic, sqhk, sqhk, sqhc,
      sbqk, sbqk, sbqc)

    attf = att.reshape(B * R * L, 2 * D)
    atto = pl.kernel(
        _sc_restore_body,
        mesh=mesh,
        compiler_params=pltpu.CompilerParams(needs_layout_passes=False),
        out_type=jax.ShapeDtypeStruct((B, R, L, 2 * D), jnp.float32),
        scratch_types=[
            pltpu.VMEM((CHUNK // 128, 128), jnp.int32),
            pltpu.VMEM((CHUNK, 2 * D), jnp.float32),
            pltpu.SemaphoreType.DMA,
        ],
    )(attf, oig4.reshape(B, R, L // 128, 128))

    out = pl.pallas_call(
        _combine_body,
        grid=(B,),
        in_specs=[
            pl.BlockSpec((1, R, L, 2 * D), lambda b: (b, 0, 0, 0)),
        ],
        out_specs=pl.BlockSpec((1, L, D), lambda b: (b, 0, 0)),
        out_shape=jax.ShapeDtypeStruct((B, L, D), jnp.float32),
    )(atto)
    return out


# speedup vs baseline: 1.0285x; 1.0285x over previous
import functools
import math

import jax
import jax.numpy as jnp
from jax import lax
from jax.experimental import pallas as pl
from jax.experimental.pallas import tpu as pltpu
from jax.experimental.pallas import tpu_sc as plsc

D_KK = 64
N_ROUNDS = 4
BKT = 64
SEQ = 4096
NBKT = SEQ // BKT
NBB = 8
NBLK = NBKT // NBB
CHUNK = 512


def _hash_rank_body(qref, rmref, oiref, oigref, sqhref, sbpref):
    q = qref[0]
    rm = rmref[0, 0]
    b = pl.program_id(0)
    r = pl.program_id(1)
    nrm = jnp.sqrt(jnp.sum(q * q, axis=1, keepdims=True))
    qn = q / jnp.maximum(nrm, 1e-12)
    m = jax.lax.dot_general(
        qn, rm, (((1,), (0,)), ((), ())), preferred_element_type=jnp.float32)

    L = SEQ
    lane64 = jax.lax.broadcasted_iota(jnp.int32, (L, NBKT), 1)
    row_iota = jax.lax.broadcasted_iota(jnp.int32, (L, 1), 0)
    lane64h = jax.lax.broadcasted_iota(jnp.int32, (1, NBKT), 1)
    lt = (jax.lax.broadcasted_iota(jnp.int32, (128, 128), 0) >=
          jax.lax.broadcasted_iota(jnp.int32, (128, 128), 1)
          ).astype(jnp.float32)
    c = jnp.concatenate([m, -m], axis=1)
    mx = jnp.max(c, axis=1, keepdims=True)
    hv = jnp.min(jnp.where(c == mx, lane64, NBKT), axis=1,
                 keepdims=True)

    onehot = (hv == lane64).astype(jnp.float32)
    pieces = []
    carry = jnp.zeros((1, NBKT), jnp.float32)
    for ch in range(L // 128):
        oh_c = onehot[ch * 128:(ch + 1) * 128, :]
        within = jax.lax.dot_general(
            lt, oh_c, (((1,), (0,)), ((), ())),
            preferred_element_type=jnp.float32)
        pieces.append(within + carry)
        carry = carry + within[127:128, :]
    ccum = jnp.concatenate(pieces, axis=0)
    c_incl = jnp.sum(onehot * ccum, axis=1, keepdims=True)
    tot = carry
    cumt = tot
    k = 1
    while k < NBKT:
        sh2 = jnp.where(lane64h < k, 0.0, jnp.roll(cumt, k, axis=1))
        cumt = cumt + sh2
        k *= 2
    offs = cumt - tot
    oi_f = jnp.sum(onehot * offs, axis=1, keepdims=True) + c_incl - 1.0
    oi = oi_f.astype(jnp.int32)

    sqh = jnp.sum(
        (cumt <= row_iota.astype(jnp.float32)).astype(jnp.int32),
        axis=1, keepdims=True)

    oiref[0, 0] = oi
    oigref[0, 0] = oi + (b * N_ROUNDS + r) * L
    sqhref[0, 0] = sqh
    part = (oi >> 6) << (8 * r)
    sbpref[0] = jnp.where(r == 0, part, sbpref[0] | part)


def _sc_reorder_body(qv_hbm, oi_hbm, oig_hbm, sbp_hbm,
                     rqv_hbm, qi_hbm, sbq_hbm,
                     oi_v, idx_v, qvbuf, qi_b, sbq_b, sbp_v, sem):
    info = plsc.get_sparse_core_info()
    b = lax.axis_index("s") * info.num_cores + lax.axis_index("c")
    L = SEQ
    pltpu.sync_copy(sbp_hbm.at[b], sbp_v)
    for r in range(N_ROUNDS):
        pltpu.sync_copy(oi_hbm.at[b, r], oi_v)

        def body(i, carry):
            sl = pl.ds(i * 16, 16)
            idx16 = oi_v[sl]
            lvals = i * 16 + lax.iota(jnp.int32, 16)
            plsc.store_scatter(qi_b, [idx16], lvals)
            plsc.store_scatter(sbq_b, [idx16], sbp_v[sl])
            return carry

        lax.fori_loop(0, L // 16, body, 0)
        pltpu.sync_copy(qi_b, qi_hbm.at[b, r])
        pltpu.sync_copy(sbq_b, sbq_hbm.at[b, r])
    nsub = CHUNK // 128
    for c in range(L // CHUNK):
        pltpu.sync_copy(qv_hbm.at[b, pl.ds(c * CHUNK, CHUNK)], qvbuf)
        for r in range(N_ROUNDS):
            pltpu.sync_copy(oig_hbm.at[b, r, pl.ds(c * nsub, nsub)], idx_v)
            for j in range(nsub):
                pltpu.async_copy(qvbuf.at[pl.ds(j * 128, 128)],
                                 rqv_hbm.at[idx_v.at[j]], sem).wait()


def _sc_restore_body(att_hbm, oig_hbm, atto_hbm, idx_v, rows_v, sem):
    info = plsc.get_sparse_core_info()
    b = lax.axis_index("s") * info.num_cores + lax.axis_index("c")
    L = SEQ
    nsub = CHUNK // 128
    for r in range(N_ROUNDS):
        for c in range(L // CHUNK):
            pltpu.sync_copy(oig_hbm.at[b, r, pl.ds(c * nsub, nsub)], idx_v)
            for j in range(nsub):
                pltpu.async_copy(att_hbm.at[idx_v.at[j]],
                                 rows_v.at[pl.ds(j * 128, 128)], sem).wait()
            pltpu.sync_copy(rows_v, atto_hbm.at[b, r, pl.ds(c * CHUNK, CHUNK)])


def _attn_body(rqv, rqvh, qik, qikh, qic, sqhk, sqhkh, sqhc,
               sbqk, sbqkh, sbqc, att):
    qv = rqv[0, 0]
    qvh = rqvh[0, 0]
    q = qv[:, :D_KK]
    qh = qvh[:, :D_KK]
    v = qv[:, D_KK:]
    vh = qvh[:, D_KK:]
    qi_rows = qik[0, 0, :, 0]
    qi_prev0 = qikh[0, 0, 0]
    qi_col = qic[0, 0]
    sq_rows = sqhk[0, 0, :, 0]
    sq_prev0 = sqhkh[0, 0, 0]
    sq_col = sqhc[0, 0]
    sb_rows = sbqk[0, 0, :, 0]
    sb_prev0 = sbqkh[0, 0, 0]
    sb_col = sbqc[0, 0]

    inv_sqrt_d = 1.0 / math.sqrt(D_KK)
    for t in range(NBB):
        qt = q[t * BKT:(t + 1) * BKT, :]
        prev = qh if t == 0 else q[(t - 1) * BKT:t * BKT, :]
        kw = jnp.concatenate([prev, qt], axis=0)
        nrm = jnp.sqrt(jnp.sum(kw * kw, axis=1, keepdims=True))
        kn = kw / jnp.maximum(nrm, 1e-12)
        s = jax.lax.dot_general(
            qt, kn, (((1,), (1,)), ((), ())),
            preferred_element_type=jnp.float32) * inv_sqrt_d

        qi_q = qi_col[t * BKT:(t + 1) * BKT, :]
        qi_p = qi_prev0 if t == 0 else qi_rows[t - 1:t, :]
        qi_k = jnp.concatenate([qi_p, qi_rows[t:t + 1, :]], axis=1)
        sq_q = sq_col[t * BKT:(t + 1) * BKT, :]
        sq_p = sq_prev0 if t == 0 else sq_rows[t - 1:t, :]
        sq_k = jnp.concatenate([sq_p, sq_rows[t:t + 1, :]], axis=1)
        sb_q = sb_col[t * BKT:(t + 1) * BKT, :]
        sb_p = sb_prev0 if t == 0 else sb_rows[t - 1:t, :]
        sb_k = jnp.concatenate([sb_p, sb_rows[t:t + 1, :]], axis=1)

        s = jnp.where(sq_q != sq_k, -1e9, s)
        s = jnp.where(qi_q < qi_k, -1e9, s)
        s = jnp.where(qi_q == qi_k, -1e5, s)

        cnt = jnp.zeros(s.shape, jnp.int32)
        for rp in range(N_ROUNDS):
            aa = (sb_q >> (8 * rp)) & 255
            bb = (sb_k >> (8 * rp)) & 255
            cnt = cnt + jnp.where(((aa - bb) & (NBKT - 1)) <= 1, 1, 0)

        m = jnp.max(s, axis=1, keepdims=True)
        e = jnp.exp(s - m)
        ssum = jnp.sum(e, axis=1, keepdims=True)
        lse_t = m + jnp.log(ssum)
        p = e / (ssum * cnt.astype(jnp.float32))

        vw = jnp.concatenate([vh if t == 0 else v[(t - 1) * BKT:t * BKT, :],
                              v[t * BKT:(t + 1) * BKT, :]], axis=0)
        ot = jax.lax.dot_general(
            p, vw, (((1,), (0,)), ((), ())),
            preferred_element_type=jnp.float32)
        att[0, 0, t * BKT:(t + 1) * BKT, :] = jnp.concatenate(
            [ot, lse_t, jnp.zeros((BKT, D_KK - 1), jnp.float32)], axis=1)


def _combine_body(att_ref, out_ref):
    a = att_ref[0]
    acc = None
    for r in range(N_ROUNDS):
        x = a[r, :, D_KK:D_KK + 1]
        m = jnp.max(x)
        e = jnp.exp(x - m)
        w = e / jnp.sum(e)
        term = a[r, :, :D_KK] * w
        acc = term if acc is None else acc + term
    out_ref[0] = acc


@jax.jit
def kernel(query, value, rand_matrix, seed):
    B, L, D = query.shape
    R = N_ROUNDS

    rm2 = rand_matrix.reshape(B, D, R, NBKT // 2).transpose(0, 2, 1, 3)
    oi4, oig4, sqh4, sbp3 = pl.pallas_call(
        _hash_rank_body,
        grid=(B, R),
        in_specs=[
            pl.BlockSpec((1, L, D), lambda b, r: (b, 0, 0)),
            pl.BlockSpec((1, 1, D, NBKT // 2), lambda b, r: (b, r, 0, 0)),
        ],
        out_specs=[
            pl.BlockSpec((1, 1, L, 1), lambda b, r: (b, r, 0, 0)),
            pl.BlockSpec((1, 1, L, 1), lambda b, r: (b, r, 0, 0)),
            pl.BlockSpec((1, 1, L, 1), lambda b, r: (b, r, 0, 0)),
            pl.BlockSpec((1, L, 1), lambda b, r: (b, 0, 0)),
        ],
        out_shape=[
            jax.ShapeDtypeStruct((B, R, L, 1), jnp.int32),
            jax.ShapeDtypeStruct((B, R, L, 1), jnp.int32),
            jax.ShapeDtypeStruct((B, R, L, 1), jnp.int32),
            jax.ShapeDtypeStruct((B, L, 1), jnp.int32),
        ],
    )(query, rm2)

    oi3 = oi4.reshape(B, R, L)
    oig3 = oig4.reshape(B, R, L)
    sbp2 = sbp3.reshape(B, L)

    qv = jnp.concatenate([query, value], axis=-1)
    mesh = plsc.VectorSubcoreMesh(core_axis_name="c", subcore_axis_name="s")
    rqvf, qi3, sbq3 = pl.kernel(
        _sc_reorder_body,
        mesh=mesh,
        compiler_params=pltpu.CompilerParams(needs_layout_passes=False),
        out_type=[
            jax.ShapeDtypeStruct((B * R * L, 2 * D), jnp.float32),
            jax.ShapeDtypeStruct((B, R, L), jnp.int32),
            jax.ShapeDtypeStruct((B, R, L), jnp.int32),
        ],
        scratch_types=[
            pltpu.VMEM((L,), jnp.int32),
            pltpu.VMEM((CHUNK // 128, 128), jnp.int32),
            pltpu.VMEM((CHUNK, 2 * D), jnp.float32),
            pltpu.VMEM((L,), jnp.int32),
            pltpu.VMEM((L,), jnp.int32),
            pltpu.VMEM((L,), jnp.int32),
            pltpu.SemaphoreType.DMA,
        ],
    )(qv, oi3, oig4.reshape(B, R, L // 128, 128), sbp2)

    rqvg = rqvf.reshape(B, R, L, 2 * D)

    qik = qi3.reshape(B, R, NBKT, 1, BKT)
    sqhk = sqh4.reshape(B, R, NBKT, 1, BKT)
    sbqk = sbq3.reshape(B, R, NBKT, 1, BKT)
    qic = qi3.reshape(B, R, L, 1)
    sqhc = sqh4.reshape(B, R, L, 1)
    sbqc = sbq3.reshape(B, R, L, 1)

    grid = (B, R, NBLK)

    def blk(b, r, n):
        return (b, r, n, 0)

    def blk_halo(b, r, n):
        return (b, r, (n * NBB - 1) % NBKT, 0)

    def blk5(b, r, n):
        return (b, r, n, 0, 0)

    def blk5_halo(b, r, n):
        return (b, r, (n * NBB - 1) % NBKT, 0, 0)

    att = pl.pallas_call(
        _attn_body,
        grid=grid,
        in_specs=[
            pl.BlockSpec((1, 1, NBB * BKT, 2 * D), blk),
            pl.BlockSpec((1, 1, BKT, 2 * D), blk_halo),
            pl.BlockSpec((1, 1, NBB, 1, BKT), blk5),
            pl.BlockSpec((1, 1, 1, 1, BKT), blk5_halo),
            pl.BlockSpec((1, 1, NBB * BKT, 1), blk),
            pl.BlockSpec((1, 1, NBB, 1, BKT), blk5),
            pl.BlockSpec((1, 1, 1, 1, BKT), blk5_halo),
            pl.BlockSpec((1, 1, NBB * BKT, 1), blk),
            pl.BlockSpec((1, 1, NBB, 1, BKT), blk5),
            pl.BlockSpec((1, 1, 1, 1, BKT), blk5_halo),
            pl.BlockSpec((1, 1, NBB * BKT, 1), blk),
        ],
        out_specs=pl.BlockSpec((1, 1, NBB * BKT, 2 * D), blk),
        out_shape=jax.ShapeDtypeStruct((B, R, L, 2 * D), jnp.float32),
    )(rqvg, rqvg, qik, qik, qic, sqhk, sqhk, sqhc,
      sbqk, sbqk, sbqc)

    attf = att.reshape(B * R * L, 2 * D)
    atto = pl.kernel(
        _sc_restore_body,
        mesh=mesh,
        compiler_params=pltpu.CompilerParams(needs_layout_passes=False),
        out_type=jax.ShapeDtypeStruct((B, R, L, 2 * D), jnp.float32),
        scratch_types=[
            pltpu.VMEM((CHUNK // 128, 128), jnp.int32),
            pltpu.VMEM((CHUNK, 2 * D), jnp.float32),
            pltpu.SemaphoreType.DMA,
        ],
    )(attf, oig4.reshape(B, R, L // 128, 128))

    out = pl.pallas_call(
        _combine_body,
        grid=(B,),
        in_specs=[
            pl.BlockSpec((1, R, L, 2 * D), lambda b: (b, 0, 0, 0)),
        ],
        out_specs=pl.BlockSpec((1, L, D), lambda b: (b, 0, 0)),
        out_shape=jax.ShapeDtypeStruct((B, L, D), jnp.float32),
    )(atto)
    return out
